# baseline (speedup 1.0000x reference)
import functools

import jax
import jax.numpy as jnp
from jax import lax
from jax.experimental import pallas as pl
from jax.experimental.pallas import tpu as pltpu
from jax.experimental.pallas import tpu_sc as plsc

D_MODEL_ = 1024
D_FF_ = 2752
NE_ = 8
NTOK_ = 2048
TILE_ = 256
NT_ = 24
NSLOT_ = NT_ * TILE_
NSLOTP_ = (NT_ + 1) * TILE_
FFB_ = 688
NFF_ = D_FF_ // FFB_
MAXT_ = NTOK_ // TILE_


def _router_body(x_ref, gw_ref, pos_ref, w2_ref, toff_ref, nt_ref):
    x = x_ref[...]
    gw = gw_ref[...]
    logits = jax.lax.dot_general(
        x.astype(jnp.bfloat16), gw.astype(jnp.bfloat16), (((1,), (1,)), ((), ())),
        preferred_element_type=jnp.float32)
    m = jnp.max(logits, axis=1, keepdims=True)
    z = jnp.exp(logits - m)
    p = z / jnp.sum(z, axis=1, keepdims=True)

    iota8 = jax.lax.broadcasted_iota(jnp.int32, (NTOK_, NE_), 1)
    m0 = jnp.max(p, axis=1, keepdims=True)
    e0 = jnp.min(jnp.where(p >= m0, iota8, NE_), axis=1, keepdims=True)
    oh0 = (iota8 == e0).astype(jnp.float32)
    p1 = jnp.where(iota8 == e0, -1.0, p)
    m1 = jnp.max(p1, axis=1, keepdims=True)
    e1 = jnp.min(jnp.where(p1 >= m1, iota8, NE_), axis=1, keepdims=True)
    oh1 = (iota8 == e1).astype(jnp.float32)
    denom = m0 + m1 + 1e-6
    w0 = m0 / denom
    w1 = m1 / denom

    hist = oh0 + oh1
    incl = hist
    sh = 1
    while sh < NTOK_:
        incl = incl + jnp.concatenate(
            [jnp.zeros((sh, NE_), jnp.float32), incl[: NTOK_ - sh, :]], axis=0)
        sh *= 2
    excl = incl - hist
    rank0 = jnp.sum(excl * oh0, axis=1, keepdims=True)
    rank1 = jnp.sum(excl * oh1, axis=1, keepdims=True)

    counts = incl[NTOK_ - 1 : NTOK_, :]
    ntiles = jnp.floor((counts + (TILE_ - 1)) * (1.0 / TILE_))
    r8 = jax.lax.broadcasted_iota(jnp.int32, (NE_, NE_), 0)
    c8 = jax.lax.broadcasted_iota(jnp.int32, (NE_, NE_), 1)
    nt_col = jnp.broadcast_to(jnp.transpose(ntiles), (NE_, NE_))
    cum_t = jnp.sum(jnp.where(r8 <= c8, nt_col, 0.0), axis=0, keepdims=True)
    toff = cum_t - ntiles
    off = toff * TILE_

    pos0 = rank0 + jnp.sum(oh0 * off, axis=1, keepdims=True)
    pos1 = rank1 + jnp.sum(oh1 * off, axis=1, keepdims=True)
    pos_ref[...] = jnp.concatenate([pos0, pos1], axis=1).astype(jnp.int32)
    w2_ref[...] = jnp.concatenate([w0, w1], axis=1)

    toff_ref[...] = jnp.transpose(toff).astype(jnp.int32)
    nt_ref[...] = jnp.transpose(ntiles).astype(jnp.int32)


def _slot_body(pos_ref, w2_ref, stok_ref, sw_ref):
    pos2 = pos_ref[...]
    w2 = w2_ref[...]
    p0i = pos2[:, 0:1]
    p1i = pos2[:, 1:2]
    w0 = w2[:, 0:1]
    w1 = w2[:, 1:2]
    tokf = jax.lax.broadcasted_iota(jnp.int32, (NTOK_, 1), 0).astype(jnp.float32)
    cols0 = jnp.concatenate([tokf, w0], axis=1)
    cols1 = jnp.concatenate([tokf, w1], axis=1)
    CH = 512
    sid = jax.lax.broadcasted_iota(jnp.int32, (NTOK_, CH), 1)
    for c in range(NSLOT_ // CH):
        b0 = (sid == p0i - c * CH).astype(jnp.float32)
        b1 = (sid == p1i - c * CH).astype(jnp.float32)
        res = jax.lax.dot_general(
            b0, cols0, (((0,), (0,)), ((), ())),
            preferred_element_type=jnp.float32,
            precision=jax.lax.Precision.HIGHEST)
        res = res + jax.lax.dot_general(
            b1, cols1, (((0,), (0,)), ((), ())),
            preferred_element_type=jnp.float32,
            precision=jax.lax.Precision.HIGHEST)
        stok_ref[pl.ds(c * CH, CH), :] = jnp.floor(res[:, 0:1] + 0.5).astype(jnp.int32)
        sw_ref[pl.ds(c * CH, CH), :] = res[:, 1:2]


def _gemm_body(toff_s, nt_s, xs_ref, sw_ref, eg_ref, eu_ref, edt_ref,
               y_ref, acc):
    e = pl.program_id(0)
    f = pl.program_id(1)
    i = pl.program_id(2)
    valid = i < nt_s[e]

    @pl.when(valid)
    def _compute():
        xb = xs_ref[...].astype(jnp.bfloat16)
        wg = eg_ref[0].astype(jnp.bfloat16)
        wu = eu_ref[0].astype(jnp.bfloat16)
        wd = edt_ref[0].astype(jnp.bfloat16)
        g = jax.lax.dot_general(xb, wg, (((1,), (1,)), ((), ())),
                                preferred_element_type=jnp.float32)
        u = jax.lax.dot_general(xb, wu, (((1,), (1,)), ((), ())),
                                preferred_element_type=jnp.float32)
        h = g * (1.0 / (1.0 + jnp.exp(-g))) * u
        h = h * sw_ref[...]
        hb = h.astype(jnp.bfloat16)
        part = jax.lax.dot_general(hb, wd, (((1,), (0,)), ((), ())),
                                   preferred_element_type=jnp.float32)

        @pl.when(f == 0)
        def _():
            acc[i] = part

        @pl.when(jnp.logical_and(f > 0, f < NFF_ - 1))
        def _():
            acc[i] = acc[i] + part

        @pl.when(f == NFF_ - 1)
        def _():
            y_ref[...] = acc[i] + part


NC_ = 2
NS_ = 16
NW_ = NC_ * NS_
L_ = 16
TROWS_ = NTOK_ // NW_
TCH_ = TROWS_ // 2

_SC_MESH = plsc.VectorSubcoreMesh(
    core_axis_name="c", subcore_axis_name="s", num_cores=NC_, num_subcores=NS_)

SROWS_ = NSLOT_ // NW_
SCH_ = SROWS_ // 2


def _sc_gather_body(stok_hbm, x_hbm, xs_hbm, ia_v, ib_v, rows_v, sem):
    wid = lax.axis_index("s") * NC_ + lax.axis_index("c")
    base = wid * SROWS_
    pltpu.sync_copy(stok_hbm.at[pl.ds(base, SCH_)], ia_v)
    pltpu.sync_copy(stok_hbm.at[pl.ds(base + SCH_, SCH_)], ib_v)
    pltpu.async_copy(x_hbm.at[ia_v], rows_v, sem).wait()
    pltpu.sync_copy(rows_v, xs_hbm.at[pl.ds(base, SCH_)])
    pltpu.async_copy(x_hbm.at[ib_v], rows_v, sem).wait()
    pltpu.sync_copy(rows_v, xs_hbm.at[pl.ds(base + SCH_, SCH_)])


def _sc_combine_body(p0_hbm, p1_hbm, y_hbm, ya_hbm, yb_hbm,
                     ia_v, ib_v, ra_v, rb_v, sem):
    wid = lax.axis_index("s") * NC_ + lax.axis_index("c")
    tbase = wid * TROWS_

    for c in range(TROWS_ // TCH_):
        cb = tbase + c * TCH_
        pltpu.sync_copy(p0_hbm.at[pl.ds(cb, TCH_)], ia_v)
        pltpu.sync_copy(p1_hbm.at[pl.ds(cb, TCH_)], ib_v)
        pltpu.async_copy(y_hbm.at[ia_v], ra_v, sem).wait()
        pltpu.async_copy(y_hbm.at[ib_v], rb_v, sem).wait()
        pltpu.sync_copy(ra_v, ya_hbm.at[pl.ds(cb, TCH_)])
        pltpu.sync_copy(rb_v, yb_hbm.at[pl.ds(cb, TCH_)])


def _add_body(a_ref, b_ref, o_ref):
    o_ref[...] = a_ref[...] + b_ref[...]


def _slot_or_trash(f, i, toff_s, nt_s, e):
    return jnp.where(jnp.logical_and(f == NFF_ - 1, i < nt_s[e]),
                     toff_s[e] + i, NT_)


@functools.partial(jax.jit, static_argnames=("interpret",))
def _moe(x, gate_w, expert_gate, expert_up, expert_down, interpret=False):
    x2 = x.reshape(NTOK_, D_MODEL_)
    edt = jnp.swapaxes(expert_down, 1, 2)

    pos2, w2, toff8, nt8 = pl.pallas_call(
        _router_body,
        out_shape=[
            jax.ShapeDtypeStruct((NTOK_, 2), jnp.int32),
            jax.ShapeDtypeStruct((NTOK_, 2), jnp.float32),
            jax.ShapeDtypeStruct((NE_, 1), jnp.int32),
            jax.ShapeDtypeStruct((NE_, 1), jnp.int32),
        ],
        interpret=interpret,
    )(x2, gate_w)

    stok, sw = pl.pallas_call(
        _slot_body,
        out_shape=[
            jax.ShapeDtypeStruct((NSLOTP_, 1), jnp.int32),
            jax.ShapeDtypeStruct((NSLOTP_, 1), jnp.float32),
        ],
        interpret=interpret,
    )(pos2, w2)

    xs = pl.kernel(
        _sc_gather_body,
        out_type=jax.ShapeDtypeStruct((NSLOTP_, D_MODEL_), jnp.float32),
        mesh=_SC_MESH,
        scratch_types=[
            pltpu.VMEM((SCH_,), jnp.int32),
            pltpu.VMEM((SCH_,), jnp.int32),
            pltpu.VMEM((SCH_, D_MODEL_), jnp.float32),
            pltpu.SemaphoreType.DMA,
        ],
    )(stok.reshape(NSLOTP_), x2)

    grid_spec = pltpu.PrefetchScalarGridSpec(
        num_scalar_prefetch=2,
        grid=(NE_, NFF_, MAXT_),
        in_specs=[
            pl.BlockSpec((TILE_, D_MODEL_),
                         lambda e, f, i, toff_s, nt_s: (
                             jnp.where(i < nt_s[e], toff_s[e] + i, NT_), 0)),
            pl.BlockSpec((TILE_, 1),
                         lambda e, f, i, toff_s, nt_s: (
                             jnp.where(i < nt_s[e], toff_s[e] + i, NT_), 0)),
            pl.BlockSpec((1, FFB_, D_MODEL_), lambda e, f, i, *s: (e, f, 0)),
            pl.BlockSpec((1, FFB_, D_MODEL_), lambda e, f, i, *s: (e, f, 0)),
            pl.BlockSpec((1, FFB_, D_MODEL_), lambda e, f, i, *s: (e, f, 0)),
        ],
        out_specs=pl.BlockSpec(
            (TILE_, D_MODEL_),
            lambda e, f, i, toff_s, nt_s: (_slot_or_trash(f, i, toff_s, nt_s, e), 0)),
        scratch_shapes=[
            pltpu.VMEM((MAXT_, TILE_, D_MODEL_), jnp.float32),
        ],
    )
    y = pl.pallas_call(
        _gemm_body,
        grid_spec=grid_spec,
        out_shape=jax.ShapeDtypeStruct((NSLOTP_, D_MODEL_), jnp.float32),
        compiler_params=pltpu.CompilerParams(
            dimension_semantics=("arbitrary", "arbitrary", "arbitrary")),
        interpret=interpret,
    )(toff8.reshape(NE_), nt8.reshape(NE_),
      xs, sw, expert_gate, expert_up, edt)

    p0f = pos2[:, 0].reshape(NTOK_)
    p1f = pos2[:, 1].reshape(NTOK_)
    ya, yb = pl.kernel(
        _sc_combine_body,
        out_type=[
            jax.ShapeDtypeStruct((NTOK_, D_MODEL_), jnp.float32),
            jax.ShapeDtypeStruct((NTOK_, D_MODEL_), jnp.float32),
        ],
        mesh=_SC_MESH,
        scratch_types=[
            pltpu.VMEM((TCH_,), jnp.int32),
            pltpu.VMEM((TCH_,), jnp.int32),
            pltpu.VMEM((TCH_, D_MODEL_), jnp.float32),
            pltpu.VMEM((TCH_, D_MODEL_), jnp.float32),
            pltpu.SemaphoreType.DMA,
        ],
    )(p0f, p1f, y)

    out = pl.pallas_call(
        _add_body,
        grid=(NTOK_ // TILE_,),
        in_specs=[
            pl.BlockSpec((TILE_, D_MODEL_), lambda i: (i, 0)),
            pl.BlockSpec((TILE_, D_MODEL_), lambda i: (i, 0)),
        ],
        out_specs=pl.BlockSpec((TILE_, D_MODEL_), lambda i: (i, 0)),
        out_shape=jax.ShapeDtypeStruct((NTOK_, D_MODEL_), jnp.float32),
        interpret=interpret,
    )(ya, yb)

    return out.reshape(x.shape)


def kernel(x, gate_w, expert_gate, expert_up, expert_down):
    return _moe(x, gate_w, expert_gate, expert_up, expert_down)


# speedup vs baseline: 1.3564x; 1.3564x over previous
import functools

import jax
import jax.numpy as jnp
from jax import lax
from jax.experimental import pallas as pl
from jax.experimental.pallas import tpu as pltpu

D_MODEL_ = 1024
D_FF_ = 2752
NE_ = 8
NTOK_ = 2048
TILE_ = 256
NT_ = 24
NSLOT_ = NT_ * TILE_
NSLOTP_ = (NT_ + 1) * TILE_
FFB_ = 688
NFF_ = D_FF_ // FFB_
MAXT_ = NTOK_ // TILE_


def _router_body(x_ref, gw_ref, pos_ref, w2_ref, toff_ref, nt_ref):
    x = x_ref[...]
    gw = gw_ref[...]
    logits = jax.lax.dot_general(
        x.astype(jnp.bfloat16), gw.astype(jnp.bfloat16), (((1,), (1,)), ((), ())),
        preferred_element_type=jnp.float32)
    m = jnp.max(logits, axis=1, keepdims=True)
    z = jnp.exp(logits - m)
    p = z / jnp.sum(z, axis=1, keepdims=True)

    iota8 = jax.lax.broadcasted_iota(jnp.int32, (NTOK_, NE_), 1)
    m0 = jnp.max(p, axis=1, keepdims=True)
    e0 = jnp.min(jnp.where(p >= m0, iota8, NE_), axis=1, keepdims=True)
    oh0 = (iota8 == e0).astype(jnp.float32)
    p1 = jnp.where(iota8 == e0, -1.0, p)
    m1 = jnp.max(p1, axis=1, keepdims=True)
    e1 = jnp.min(jnp.where(p1 >= m1, iota8, NE_), axis=1, keepdims=True)
    oh1 = (iota8 == e1).astype(jnp.float32)
    denom = m0 + m1 + 1e-6
    w0 = m0 / denom
    w1 = m1 / denom

    hist = oh0 + oh1
    incl = hist
    sh = 1
    while sh < NTOK_:
        incl = incl + jnp.concatenate(
            [jnp.zeros((sh, NE_), jnp.float32), incl[: NTOK_ - sh, :]], axis=0)
        sh *= 2
    excl = incl - hist
    rank0 = jnp.sum(excl * oh0, axis=1, keepdims=True)
    rank1 = jnp.sum(excl * oh1, axis=1, keepdims=True)

    counts = incl[NTOK_ - 1 : NTOK_, :]
    ntiles = jnp.floor((counts + (TILE_ - 1)) * (1.0 / TILE_))
    r8 = jax.lax.broadcasted_iota(jnp.int32, (NE_, NE_), 0)
    c8 = jax.lax.broadcasted_iota(jnp.int32, (NE_, NE_), 1)
    nt_col = jnp.broadcast_to(jnp.transpose(ntiles), (NE_, NE_))
    cum_t = jnp.sum(jnp.where(r8 <= c8, nt_col, 0.0), axis=0, keepdims=True)
    toff = cum_t - ntiles
    off = toff * TILE_

    pos0 = rank0 + jnp.sum(oh0 * off, axis=1, keepdims=True)
    pos1 = rank1 + jnp.sum(oh1 * off, axis=1, keepdims=True)
    pos_ref[...] = jnp.concatenate([pos0, pos1], axis=1).astype(jnp.int32)
    w2_ref[...] = jnp.concatenate([w0, w1], axis=1)

    toff_ref[...] = jnp.transpose(toff).astype(jnp.int32)
    nt_ref[...] = jnp.transpose(ntiles).astype(jnp.int32)


def _slot_body(pos_ref, w2_ref, stok_ref, sw_ref):
    pos2 = pos_ref[...]
    w2 = w2_ref[...]
    tokf = jax.lax.broadcasted_iota(
        jnp.int32, (NTOK_, 1), 0).astype(jnp.float32)
    it_t = jax.lax.broadcasted_iota(jnp.int32, (NTOK_, NT_), 1)
    it_o = jax.lax.broadcasted_iota(jnp.int32, (NTOK_, TILE_), 1)
    acc_tok = jnp.zeros((NT_, TILE_), jnp.float32)
    acc_w = jnp.zeros((NT_, TILE_), jnp.float32)
    for k in range(2):
        posk = pos2[:, k : k + 1]
        tile = lax.div(posk, TILE_)
        off = posk - tile * TILE_
        m1 = (it_t == tile).astype(jnp.float32)
        m2 = (it_o == off).astype(jnp.float32)
        wk = w2[:, k : k + 1]
        acc_tok = acc_tok + jax.lax.dot_general(
            m1 * tokf, m2, (((0,), (0,)), ((), ())),
            preferred_element_type=jnp.float32,
            precision=jax.lax.Precision.HIGHEST)
        acc_w = acc_w + jax.lax.dot_general(
            m1 * wk, m2, (((0,), (0,)), ((), ())),
            preferred_element_type=jnp.float32,
            precision=jax.lax.Precision.HIGHEST)
    stok_ref[...] = jnp.floor(acc_tok + 0.5).astype(jnp.int32)
    sw_ref[...] = acc_w


def _gemm_body(toff_s, nt_s, stok_s, x_ref, sw_ref, eg_ref, eu_ref, edt_ref,
               y_ref, xg, acc):
    e = pl.program_id(0)
    f = pl.program_id(1)
    i = pl.program_id(2)
    valid = i < nt_s[e]

    @pl.when(jnp.logical_and(valid, f == 0))
    def _gather():
        base = (toff_s[e] + i) * TILE_

        def body(r, _):
            tok = stok_s[base + r]
            xg[i, pl.ds(r, 1), :] = x_ref[pl.ds(tok, 1), :]
            return 0
        jax.lax.fori_loop(0, TILE_, body, 0)

    @pl.when(valid)
    def _compute():
        xb = xg[i].astype(jnp.bfloat16)
        wg = eg_ref[0].astype(jnp.bfloat16)
        wu = eu_ref[0].astype(jnp.bfloat16)
        wd = edt_ref[0].astype(jnp.bfloat16)
        g = jax.lax.dot_general(xb, wg, (((1,), (1,)), ((), ())),
                                preferred_element_type=jnp.float32)
        u = jax.lax.dot_general(xb, wu, (((1,), (1,)), ((), ())),
                                preferred_element_type=jnp.float32)
        h = g * (1.0 / (1.0 + jnp.exp(-g))) * u
        h = h * sw_ref[...]
        hb = h.astype(jnp.bfloat16)
        part = jax.lax.dot_general(hb, wd, (((1,), (0,)), ((), ())),
                                   preferred_element_type=jnp.float32)

        @pl.when(f == 0)
        def _():
            acc[i] = part

        @pl.when(jnp.logical_and(f > 0, f < NFF_ - 1))
        def _():
            acc[i] = acc[i] + part

        @pl.when(f == NFF_ - 1)
        def _():
            y_ref[...] = acc[i] + part


def _combine_body(pos_s, y_ref, o_ref):
    i = pl.program_id(0)

    def body(r, _):
        p0 = pos_s[2 * (i * TILE_ + r)]
        p1 = pos_s[2 * (i * TILE_ + r) + 1]
        o_ref[pl.ds(r, 1), :] = y_ref[pl.ds(p0, 1), :] + y_ref[pl.ds(p1, 1), :]
        return 0

    jax.lax.fori_loop(0, TILE_, body, 0)


def _slot_or_trash(f, i, toff_s, nt_s, e):
    return jnp.where(jnp.logical_and(f == NFF_ - 1, i < nt_s[e]),
                     toff_s[e] + i, NT_)


@functools.partial(jax.jit, static_argnames=("interpret",))
def _moe(x, gate_w, expert_gate, expert_up, expert_down, interpret=False):
    x2 = x.reshape(NTOK_, D_MODEL_)
    edt = jnp.swapaxes(expert_down, 1, 2).astype(jnp.bfloat16)

    pos2, w2, toff8, nt8 = pl.pallas_call(
        _router_body,
        out_shape=[
            jax.ShapeDtypeStruct((NTOK_, 2), jnp.int32),
            jax.ShapeDtypeStruct((NTOK_, 2), jnp.float32),
            jax.ShapeDtypeStruct((NE_, 1), jnp.int32),
            jax.ShapeDtypeStruct((NE_, 1), jnp.int32),
        ],
        interpret=interpret,
    )(x2, gate_w)

    stok, sw = pl.pallas_call(
        _slot_body,
        out_shape=[
            jax.ShapeDtypeStruct((NT_, TILE_), jnp.int32),
            jax.ShapeDtypeStruct((NT_, TILE_), jnp.float32),
        ],
        interpret=interpret,
    )(pos2, w2)
    swp = jnp.concatenate(
        [sw.reshape(NSLOT_, 1), jnp.zeros((TILE_, 1), jnp.float32)], axis=0)

    grid_spec = pltpu.PrefetchScalarGridSpec(
        num_scalar_prefetch=3,
        grid=(NE_, NFF_, MAXT_),
        in_specs=[
            pl.BlockSpec((NTOK_, D_MODEL_), lambda e, f, i, *s: (0, 0)),
            pl.BlockSpec((TILE_, 1),
                         lambda e, f, i, toff_s, nt_s, st: (
                             jnp.where(i < nt_s[e], toff_s[e] + i, NT_), 0)),
            pl.BlockSpec((1, FFB_, D_MODEL_), lambda e, f, i, *s: (e, f, 0)),
            pl.BlockSpec((1, FFB_, D_MODEL_), lambda e, f, i, *s: (e, f, 0)),
            pl.BlockSpec((1, FFB_, D_MODEL_), lambda e, f, i, *s: (e, f, 0)),
        ],
        out_specs=pl.BlockSpec(
            (TILE_, D_MODEL_),
            lambda e, f, i, toff_s, nt_s, st: (_slot_or_trash(f, i, toff_s, nt_s, e), 0)),
        scratch_shapes=[
            pltpu.VMEM((MAXT_, TILE_, D_MODEL_), jnp.float32),
            pltpu.VMEM((MAXT_, TILE_, D_MODEL_), jnp.float32),
        ],
    )
    y = pl.pallas_call(
        _gemm_body,
        grid_spec=grid_spec,
        out_shape=jax.ShapeDtypeStruct((NSLOTP_, D_MODEL_), jnp.float32),
        compiler_params=pltpu.CompilerParams(
            dimension_semantics=("arbitrary", "arbitrary", "arbitrary")),
        interpret=interpret,
    )(toff8.reshape(NE_), nt8.reshape(NE_), stok.reshape(NSLOT_),
      x2, swp, expert_gate, expert_up, edt)

    out = pl.pallas_call(
        _combine_body,
        grid_spec=pltpu.PrefetchScalarGridSpec(
            num_scalar_prefetch=1,
            grid=(NTOK_ // TILE_,),
            in_specs=[pl.BlockSpec((NSLOTP_, D_MODEL_), lambda i, *s: (0, 0))],
            out_specs=pl.BlockSpec((TILE_, D_MODEL_), lambda i, *s: (i, 0)),
        ),
        out_shape=jax.ShapeDtypeStruct((NTOK_, D_MODEL_), jnp.float32),
        interpret=interpret,
    )(pos2.reshape(2 * NTOK_), y)

    return out.reshape(x.shape)


def kernel(x, gate_w, expert_gate, expert_up, expert_down):
    return _moe(x, gate_w, expert_gate, expert_up, expert_down)


# speedup vs baseline: 1.3731x; 1.0123x over previous
import functools

import jax
import jax.numpy as jnp
from jax import lax
from jax.experimental import pallas as pl
from jax.experimental.pallas import tpu as pltpu

D_MODEL_ = 1024
D_FF_ = 2752
NE_ = 8
NTOK_ = 2048
TILE_ = 256
NT_ = 24
NSLOT_ = NT_ * TILE_
NSLOTP_ = (NT_ + 1) * TILE_
FFB_ = 688
NFF_ = D_FF_ // FFB_
MAXT_ = NTOK_ // TILE_


def _router_body(x_ref, gw_ref, pos_ref, stok_ref, sw_ref, toff_ref, nt_ref):
    x = x_ref[...]
    gw = gw_ref[...]
    logits = jax.lax.dot_general(
        x.astype(jnp.bfloat16), gw.astype(jnp.bfloat16), (((1,), (1,)), ((), ())),
        preferred_element_type=jnp.float32)
    m = jnp.max(logits, axis=1, keepdims=True)
    z = jnp.exp(logits - m)
    p = z / jnp.sum(z, axis=1, keepdims=True)

    iota8 = jax.lax.broadcasted_iota(jnp.int32, (NTOK_, NE_), 1)
    m0 = jnp.max(p, axis=1, keepdims=True)
    e0 = jnp.min(jnp.where(p >= m0, iota8, NE_), axis=1, keepdims=True)
    oh0 = (iota8 == e0).astype(jnp.float32)
    p1 = jnp.where(iota8 == e0, -1.0, p)
    m1 = jnp.max(p1, axis=1, keepdims=True)
    e1 = jnp.min(jnp.where(p1 >= m1, iota8, NE_), axis=1, keepdims=True)
    oh1 = (iota8 == e1).astype(jnp.float32)
    denom = m0 + m1 + 1e-6
    w0 = m0 / denom
    w1 = m1 / denom

    hist = oh0 + oh1
    incl = hist
    sh = 1
    while sh < NTOK_:
        incl = incl + jnp.concatenate(
            [jnp.zeros((sh, NE_), jnp.float32), incl[: NTOK_ - sh, :]], axis=0)
        sh *= 2
    excl = incl - hist
    rank0 = jnp.sum(excl * oh0, axis=1, keepdims=True)
    rank1 = jnp.sum(excl * oh1, axis=1, keepdims=True)

    counts = incl[NTOK_ - 1 : NTOK_, :]
    ntiles = jnp.floor((counts + (TILE_ - 1)) * (1.0 / TILE_))
    r8 = jax.lax.broadcasted_iota(jnp.int32, (NE_, NE_), 0)
    c8 = jax.lax.broadcasted_iota(jnp.int32, (NE_, NE_), 1)
    nt_col = jnp.broadcast_to(jnp.transpose(ntiles), (NE_, NE_))
    cum_t = jnp.sum(jnp.where(r8 <= c8, nt_col, 0.0), axis=0, keepdims=True)
    toff = cum_t - ntiles
    off = toff * TILE_

    pos0 = rank0 + jnp.sum(oh0 * off, axis=1, keepdims=True)
    pos1 = rank1 + jnp.sum(oh1 * off, axis=1, keepdims=True)
    pos_ref[...] = jnp.concatenate([pos0, pos1], axis=1).astype(jnp.int32)

    toff_ref[...] = jnp.transpose(toff).astype(jnp.int32)
    nt_ref[...] = jnp.transpose(ntiles).astype(jnp.int32)

    tokf = jax.lax.broadcasted_iota(
        jnp.int32, (NTOK_, 1), 0).astype(jnp.float32)
    it_t = jax.lax.broadcasted_iota(jnp.int32, (NTOK_, NT_), 1)
    it_o = jax.lax.broadcasted_iota(jnp.int32, (NTOK_, TILE_), 1)
    acc_tok = jnp.zeros((NT_, TILE_), jnp.float32)
    acc_w = jnp.zeros((NT_, TILE_), jnp.float32)
    for posk_f, wk in ((pos0, w0), (pos1, w1)):
        posk = posk_f.astype(jnp.int32)
        tile = lax.div(posk, TILE_)
        off = posk - tile * TILE_
        m1 = (it_t == tile).astype(jnp.float32)
        m2 = (it_o == off).astype(jnp.float32)
        acc_tok = acc_tok + jax.lax.dot_general(
            m1 * tokf, m2, (((0,), (0,)), ((), ())),
            preferred_element_type=jnp.float32,
            precision=jax.lax.Precision.HIGHEST)
        acc_w = acc_w + jax.lax.dot_general(
            m1 * wk, m2, (((0,), (0,)), ((), ())),
            preferred_element_type=jnp.float32,
            precision=jax.lax.Precision.HIGHEST)
    stok_ref[...] = jnp.floor(acc_tok + 0.5).astype(jnp.int32)
    sw_ref[...] = acc_w


def _gemm_body(toff_s, nt_s, stok_s, x_ref, sw_ref, eg_ref, eu_ref, edt_ref,
               y_ref, xg, acc):
    e = pl.program_id(0)
    f = pl.program_id(1)
    i = pl.program_id(2)
    valid = i < nt_s[e]

    @pl.when(jnp.logical_and(valid, f == 0))
    def _gather():
        base = (toff_s[e] + i) * TILE_

        def body(r, _):
            tok = stok_s[base + r]
            xg[i, pl.ds(r, 1), :] = x_ref[pl.ds(tok, 1), :]
            return 0
        jax.lax.fori_loop(0, TILE_, body, 0)

    @pl.when(valid)
    def _compute():
        xb = xg[i].astype(jnp.bfloat16)
        wg = eg_ref[0].astype(jnp.bfloat16)
        wu = eu_ref[0].astype(jnp.bfloat16)
        wd = edt_ref[0].astype(jnp.bfloat16)
        g = jax.lax.dot_general(xb, wg, (((1,), (1,)), ((), ())),
                                preferred_element_type=jnp.float32)
        u = jax.lax.dot_general(xb, wu, (((1,), (1,)), ((), ())),
                                preferred_element_type=jnp.float32)
        h = g * (1.0 / (1.0 + jnp.exp(-g))) * u
        h = h * sw_ref[...]
        hb = h.astype(jnp.bfloat16)
        part = jax.lax.dot_general(hb, wd, (((1,), (0,)), ((), ())),
                                   preferred_element_type=jnp.float32)

        @pl.when(f == 0)
        def _():
            acc[i] = part

        @pl.when(jnp.logical_and(f > 0, f < NFF_ - 1))
        def _():
            acc[i] = acc[i] + part

        @pl.when(f == NFF_ - 1)
        def _():
            y_ref[...] = acc[i] + part


def _combine_body(pos_s, y_ref, o_ref):
    i = pl.program_id(0)

    def body(r, _):
        p0 = pos_s[2 * (i * TILE_ + r)]
        p1 = pos_s[2 * (i * TILE_ + r) + 1]
        o_ref[pl.ds(r, 1), :] = y_ref[pl.ds(p0, 1), :] + y_ref[pl.ds(p1, 1), :]
        return 0

    jax.lax.fori_loop(0, TILE_, body, 0)


def _slot_or_trash(f, i, toff_s, nt_s, e):
    return jnp.where(jnp.logical_and(f == NFF_ - 1, i < nt_s[e]),
                     toff_s[e] + i, NT_)


@functools.partial(jax.jit, static_argnames=("interpret",))
def _moe(x, gate_w, expert_gate, expert_up, expert_down, interpret=False):
    x2 = x.reshape(NTOK_, D_MODEL_)
    edt = jnp.swapaxes(expert_down, 1, 2).astype(jnp.bfloat16)

    pos2, stok, sw, toff8, nt8 = pl.pallas_call(
        _router_body,
        out_shape=[
            jax.ShapeDtypeStruct((NTOK_, 2), jnp.int32),
            jax.ShapeDtypeStruct((NT_, TILE_), jnp.int32),
            jax.ShapeDtypeStruct((NT_, TILE_), jnp.float32),
            jax.ShapeDtypeStruct((NE_, 1), jnp.int32),
            jax.ShapeDtypeStruct((NE_, 1), jnp.int32),
        ],
        interpret=interpret,
    )(x2, gate_w)
    swp = jnp.concatenate(
        [sw.reshape(NSLOT_, 1), jnp.zeros((TILE_, 1), jnp.float32)], axis=0)

    grid_spec = pltpu.PrefetchScalarGridSpec(
        num_scalar_prefetch=3,
        grid=(NE_, NFF_, MAXT_),
        in_specs=[
            pl.BlockSpec((NTOK_, D_MODEL_), lambda e, f, i, *s: (0, 0)),
            pl.BlockSpec((TILE_, 1),
                         lambda e, f, i, toff_s, nt_s, st: (
                             jnp.where(i < nt_s[e], toff_s[e] + i, NT_), 0)),
            pl.BlockSpec((1, FFB_, D_MODEL_), lambda e, f, i, *s: (e, f, 0)),
            pl.BlockSpec((1, FFB_, D_MODEL_), lambda e, f, i, *s: (e, f, 0)),
            pl.BlockSpec((1, FFB_, D_MODEL_), lambda e, f, i, *s: (e, f, 0)),
        ],
        out_specs=pl.BlockSpec(
            (TILE_, D_MODEL_),
            lambda e, f, i, toff_s, nt_s, st: (_slot_or_trash(f, i, toff_s, nt_s, e), 0)),
        scratch_shapes=[
            pltpu.VMEM((MAXT_, TILE_, D_MODEL_), jnp.float32),
            pltpu.VMEM((MAXT_, TILE_, D_MODEL_), jnp.float32),
        ],
    )
    y = pl.pallas_call(
        _gemm_body,
        grid_spec=grid_spec,
        out_shape=jax.ShapeDtypeStruct((NSLOTP_, D_MODEL_), jnp.float32),
        compiler_params=pltpu.CompilerParams(
            dimension_semantics=("arbitrary", "arbitrary", "arbitrary")),
        interpret=interpret,
    )(toff8.reshape(NE_), nt8.reshape(NE_), stok.reshape(NSLOT_),
      x2, swp, expert_gate, expert_up, edt)

    out = pl.pallas_call(
        _combine_body,
        grid_spec=pltpu.PrefetchScalarGridSpec(
            num_scalar_prefetch=1,
            grid=(NTOK_ // TILE_,),
            in_specs=[pl.BlockSpec((NSLOTP_, D_MODEL_), lambda i, *s: (0, 0))],
            out_specs=pl.BlockSpec((TILE_, D_MODEL_), lambda i, *s: (i, 0)),
        ),
        out_shape=jax.ShapeDtypeStruct((NTOK_, D_MODEL_), jnp.float32),
        interpret=interpret,
    )(pos2.reshape(2 * NTOK_), y)

    return out.reshape(x.shape)


def kernel(x, gate_w, expert_gate, expert_up, expert_down):
    return _moe(x, gate_w, expert_gate, expert_up, expert_down)
